# baseline (speedup 1.0000x reference)
import functools

import jax
import jax.numpy as jnp
from jax import lax
from jax.experimental import pallas as pl
from jax.experimental.pallas import tpu as pltpu

_BS = 200
_K = 4
_SUB = 2


def _gcn_kernel(
    x_ref, w1_ref, b1_ref, w2_ref, b2_ref, adj_hbm,
    o_ref, s1_scr, s2_scr, ring, sems, *, bs: int, nstep: int, k: int, sub: int
):
    p = pl.program_id(0)
    i = pl.program_id(1)
    g = p * nstep + i
    sbs = bs // sub

    def _copy(t, j):
        r = lax.rem(t, nstep)
        slot = lax.rem(t, k)
        return pltpu.make_async_copy(
            adj_hbm.at[pl.ds(r * bs + j * sbs, sbs), :],
            ring.at[slot, pl.ds(j * sbs, sbs), :],
            sems.at[slot, j],
        )

    def _issue(t):
        for j in range(sub):
            _copy(t, j).start()

    @pl.when(g == 0)
    def _():
        s1_scr[...] = jnp.dot(
            x_ref[...], w1_ref[...], preferred_element_type=jnp.float32
        )
        for t in range(k):
            _issue(t)

    @pl.when(g > 0)
    def _():
        t = g + k - 1

        @pl.when(t < 2 * nstep)
        def _():
            _issue(t)

    slot = lax.rem(g, k)
    for j in range(sub):
        _copy(g, j).wait()
    a = ring[slot]

    @pl.when(p == 0)
    def _():
        acc = jnp.dot(a, s1_scr[...], preferred_element_type=jnp.float32)
        h = jnp.maximum(acc + b1_ref[...], 0.0)
        s2 = jnp.dot(h, w2_ref[...], preferred_element_type=jnp.float32)
        s2_scr[pl.ds(i * bs, bs), :] = s2
        o_ref[...] = s2

    @pl.when(p == 1)
    def _():
        acc = jnp.dot(a, s2_scr[...], preferred_element_type=jnp.float32)
        o_ref[...] = acc + b2_ref[...]


def kernel(x, adj, W1, b1, W2, b2):
    n, nfeat = x.shape
    nhid = W1.shape[1]
    nout = W2.shape[1]
    bs = _BS if n % _BS == 0 else n
    k = _K if n != bs else 1
    sub = _SUB if (bs // _SUB) % 8 == 0 and bs % _SUB == 0 else 1
    nstep = n // bs

    b1r = b1.reshape(1, nhid)
    b2r = b2.reshape(1, nout)

    return pl.pallas_call(
        functools.partial(_gcn_kernel, bs=bs, nstep=nstep, k=k, sub=sub),
        grid=(2, nstep),
        in_specs=[
            pl.BlockSpec((n, nfeat), lambda p, i: (0, 0)),
            pl.BlockSpec((nfeat, nhid), lambda p, i: (0, 0)),
            pl.BlockSpec((1, nhid), lambda p, i: (0, 0)),
            pl.BlockSpec((nhid, nout), lambda p, i: (0, 0)),
            pl.BlockSpec((1, nout), lambda p, i: (0, 0)),
            pl.BlockSpec(memory_space=pl.ANY),
        ],
        out_specs=pl.BlockSpec((bs, nout), lambda p, i: (i, 0)),
        out_shape=jax.ShapeDtypeStruct((n, nout), jnp.float32),
        scratch_shapes=[
            pltpu.VMEM((n, nhid), jnp.float32),
            pltpu.VMEM((n, nhid), jnp.float32),
            pltpu.VMEM((k, bs, n), jnp.float32),
            pltpu.SemaphoreType.DMA((k, sub)),
        ],
        compiler_params=pltpu.CompilerParams(
            dimension_semantics=("arbitrary", "arbitrary"),
        ),
    )(x, W1, b1r, W2, b2r, adj)


# speedup vs baseline: 1.0094x; 1.0094x over previous
import functools

import jax
import jax.numpy as jnp
from jax import lax
from jax.experimental import pallas as pl
from jax.experimental.pallas import tpu as pltpu

_BS = 80
_K = 6
_SUB = 2


def _gcn_kernel(
    x_ref, w1_ref, b1_ref, w2_ref, b2_ref, adj_hbm,
    o_ref, s1_scr, s2_scr, ring, sems, *, bs: int, nstep: int, k: int, sub: int
):
    p = pl.program_id(0)
    i = pl.program_id(1)
    g = p * nstep + i
    sbs = bs // sub

    def _copy(t, j):
        r = lax.rem(t, nstep)
        slot = lax.rem(t, k)
        return pltpu.make_async_copy(
            adj_hbm.at[pl.ds(r * bs + j * sbs, sbs), :],
            ring.at[slot, pl.ds(j * sbs, sbs), :],
            sems.at[slot, j],
        )

    def _issue(t):
        for j in range(sub):
            _copy(t, j).start()

    @pl.when(g == 0)
    def _():
        s1_scr[...] = jnp.dot(
            x_ref[...], w1_ref[...], preferred_element_type=jnp.float32
        ).astype(jnp.bfloat16)
        for t in range(k):
            _issue(t)

    @pl.when(g > 0)
    def _():
        t = g + k - 1

        @pl.when(t < 2 * nstep)
        def _():
            _issue(t)

    slot = lax.rem(g, k)
    for j in range(sub):
        _copy(g, j).wait()
    a = ring[slot].astype(jnp.bfloat16)

    @pl.when(p == 0)
    def _():
        acc = jnp.dot(a, s1_scr[...], preferred_element_type=jnp.float32)
        h = jnp.maximum(acc + b1_ref[...], 0.0).astype(jnp.bfloat16)
        s2 = jnp.dot(h, w2_ref[...].astype(jnp.bfloat16), preferred_element_type=jnp.float32)
        s2_scr[pl.ds(i * bs, bs), :] = s2.astype(jnp.bfloat16)
        o_ref[...] = s2

    @pl.when(p == 1)
    def _():
        acc = jnp.dot(a, s2_scr[...], preferred_element_type=jnp.float32)
        o_ref[...] = acc + b2_ref[...]


def kernel(x, adj, W1, b1, W2, b2):
    n, nfeat = x.shape
    nhid = W1.shape[1]
    nout = W2.shape[1]
    bs = _BS if n % _BS == 0 else n
    k = _K if n != bs else 1
    sub = _SUB if (bs // _SUB) % 8 == 0 and bs % _SUB == 0 else 1
    nstep = n // bs

    b1r = b1.reshape(1, nhid)
    b2r = b2.reshape(1, nout)

    return pl.pallas_call(
        functools.partial(_gcn_kernel, bs=bs, nstep=nstep, k=k, sub=sub),
        grid=(2, nstep),
        in_specs=[
            pl.BlockSpec((n, nfeat), lambda p, i: (0, 0)),
            pl.BlockSpec((nfeat, nhid), lambda p, i: (0, 0)),
            pl.BlockSpec((1, nhid), lambda p, i: (0, 0)),
            pl.BlockSpec((nhid, nout), lambda p, i: (0, 0)),
            pl.BlockSpec((1, nout), lambda p, i: (0, 0)),
            pl.BlockSpec(memory_space=pl.ANY),
        ],
        out_specs=pl.BlockSpec((bs, nout), lambda p, i: (i, 0)),
        out_shape=jax.ShapeDtypeStruct((n, nout), jnp.float32),
        scratch_shapes=[
            pltpu.VMEM((n, nhid), jnp.bfloat16),
            pltpu.VMEM((n, nhid), jnp.bfloat16),
            pltpu.VMEM((k, bs, n), jnp.float32),
            pltpu.SemaphoreType.DMA((k, sub)),
        ],
        compiler_params=pltpu.CompilerParams(
            dimension_semantics=("arbitrary", "arbitrary"),
        ),
    )(x, W1, b1r, W2, b2r, adj)
